# baseline (speedup 1.0000x reference)
import functools
import math

import jax
import jax.numpy as jnp
from jax import lax
from jax.experimental import pallas as pl
from jax.experimental.pallas import tpu as pltpu
from jax.experimental.pallas import tpu_sc as plsc

D_MODEL = 512
EMB_SCALE = math.sqrt(float(D_MODEL))


def _sc_gather(table, idx_flat, n_idx, d):
    info = plsc.get_sparse_core_info()
    num_workers = info.num_cores * info.num_subcores
    per_worker = n_idx // num_workers
    chunk = 200
    n_chunks = per_worker // chunk
    mesh = plsc.VectorSubcoreMesh(core_axis_name="c", subcore_axis_name="s")

    @functools.partial(
        pl.kernel,
        mesh=mesh,
        out_type=jax.ShapeDtypeStruct((n_idx, d), jnp.float32),
        scratch_types=[
            pltpu.VMEM((chunk,), jnp.int32),
            pltpu.VMEM((chunk, d), jnp.float32),
            pltpu.SemaphoreType.DMA,
        ],
    )
    def k(table_hbm, idx_hbm, out_hbm, idx_v, rows_v, sem):
        wid = lax.axis_index("s") * info.num_cores + lax.axis_index("c")
        base = wid * per_worker
        for c in range(n_chunks):
            off = base + c * chunk
            pltpu.sync_copy(idx_hbm.at[pl.ds(off, chunk)], idx_v)
            pltpu.async_copy(table_hbm.at[idx_v], rows_v, sem).wait()
            pltpu.sync_copy(rows_v, out_hbm.at[pl.ds(off, chunk)])

    return k(table, idx_flat)


def _tc_fuse(gathered, spec, w, b):
    bsz, t, d = spec.shape
    f = gathered.shape[1]
    bb = 8
    grid = (bsz // bb,)

    def body(g_ref, s_ref, w_ref, b_ref, o_ref):
        g = g_ref[...] * EMB_SCALE
        s2 = s_ref[...].reshape(bb * t, d)
        m = jnp.dot(s2, w_ref[...], preferred_element_type=jnp.float32)
        m = (m + b_ref[...]).reshape(bb, t, d)
        o_ref[...] = jnp.concatenate([g, m], axis=1)

    return pl.pallas_call(
        body,
        grid=grid,
        in_specs=[
            pl.BlockSpec((bb, f, d), lambda i: (i, 0, 0)),
            pl.BlockSpec((bb, t, d), lambda i: (i, 0, 0)),
            pl.BlockSpec((d, d), lambda i: (0, 0)),
            pl.BlockSpec((1, d), lambda i: (0, 0)),
        ],
        out_specs=pl.BlockSpec((bb, f + t, d), lambda i: (i, 0, 0)),
        out_shape=jax.ShapeDtypeStruct((bsz, f + t, d), jnp.float32),
    )(gathered, spec, w, b)


def kernel(formula, spec, formula_table, W_spec, b_spec):
    bsz, f = formula.shape
    idx = formula.reshape(-1).astype(jnp.int32)
    gathered = _sc_gather(formula_table, idx, bsz * f, D_MODEL)
    gathered = gathered.reshape(bsz, f, D_MODEL)
    return _tc_fuse(gathered, spec, W_spec, b_spec.reshape(1, D_MODEL))
